# baseline (speedup 1.0000x reference)
import functools

import jax
import jax.numpy as jnp
from jax import lax
from jax.experimental import pallas as pl
from jax.experimental.pallas import tpu as pltpu
from jax.experimental.pallas import tpu_sc as plsc

N_NODES = 10000
N_PAD = 10240
N_EDGES = 320000
D = 128
L = 16
NC, NS = 2, 16
NW = NC * NS
E_PER_W = N_EDGES // NW
CH = 128
NB = 3
NFULL = E_PER_W // CH
TAIL = E_PER_W - NFULL * CH
ROWS_PER_TILE = N_PAD // NS
CR = 64
R_PER_W = N_PAD // NW
NSC = 81


def _wid():
  return lax.axis_index("s") * NC + lax.axis_index("c")


@functools.cache
def _build():
  mesh = plsc.VectorSubcoreMesh(
      core_axis_name="c", subcore_axis_name="s", num_cores=NC, num_subcores=NS)

  @functools.partial(
      pl.kernel,
      out_type=jax.ShapeDtypeStruct((NC, N_PAD, D), jnp.float32),
      mesh=mesh,
      compiler_params=pltpu.CompilerParams(needs_layout_passes=False),
      scratch_types=[
          pltpu.VMEM_SHARED((N_PAD, D), jnp.float32),
          pltpu.VMEM((E_PER_W,), jnp.int32),
          [pltpu.VMEM((CH, D), jnp.float32)] * 2,
          [pltpu.VMEM((CH,), jnp.int32)] * 2,
          [pltpu.VMEM((CH,), jnp.float32)] * 2,
          pltpu.VMEM((TAIL, D), jnp.float32),
          pltpu.VMEM((TAIL,), jnp.int32),
          pltpu.VMEM((TAIL,), jnp.float32),
          [pltpu.SemaphoreType.DMA] * 2,
          [pltpu.SemaphoreType.DMA] * 2,
          [pltpu.SemaphoreType.DMA] * 2,
          pltpu.SemaphoreType.DMA,
      ],
  )
  def accum(x_hbm, src_hbm, dst_hbm, attr_hbm, zero_hbm, part_hbm,
            acc_sh, s_all, rows, dib, av, rows_t, di_t, av_t,
            sg, sm, ss, sem_t):
    cid = lax.axis_index("c")
    sid = lax.axis_index("s")
    base_e = _wid() * E_PER_W

    r0 = sid * ROWS_PER_TILE
    pltpu.sync_copy(zero_hbm.at[pl.ds(r0, ROWS_PER_TILE)],
                    acc_sh.at[pl.ds(r0, ROWS_PER_TILE)])
    pltpu.sync_copy(src_hbm.at[pl.ds(base_e, E_PER_W)], s_all)
    plsc.subcore_barrier()

    def fetch(k, b):
      pltpu.async_copy(dst_hbm.at[pl.ds(base_e + k * CH, CH)], dib[b], sm[b])
      pltpu.async_copy(attr_hbm.at[pl.ds(base_e + k * CH, CH)], av[b], sm[b])
      pltpu.async_copy(x_hbm.at[s_all.at[pl.ds(k * CH, CH)]], rows[b], sg[b])

    def wait_fetch(k, b):
      pltpu.make_async_copy(
          dst_hbm.at[pl.ds(base_e + k * CH, CH)], dib[b], sm[b]).wait()
      pltpu.make_async_copy(
          attr_hbm.at[pl.ds(base_e + k * CH, CH)], av[b], sm[b]).wait()
      pltpu.make_async_copy(
          x_hbm.at[s_all.at[pl.ds(k * CH, CH)]], rows[b], sg[b]).wait()

    def scale_rows(buf, attr_buf, n):
      @pl.loop(0, n)
      def _scale(e):
        w = plsc.load_gather(attr_buf, [jnp.zeros((L,), jnp.int32) + e])
        for dd in range(D // L):
          sl = pl.ds(dd * L, L)
          buf[e, sl] = buf[e, sl] * w

    fetch(0, 0)

    @pl.loop(0, NFULL // 2)
    def _outer(i):
      for b in range(2):
        k = i * 2 + b
        bn = 1 - b

        @pl.when(k >= 1)
        def _retire():
          pltpu.make_async_copy(rows[bn], acc_sh.at[dib[bn]], ss[bn]).wait()

        @pl.when(k + 1 < NFULL)
        def _refill():
          fetch(k + 1, bn)

        wait_fetch(k, b)
        scale_rows(rows[b], av[b], CH)
        pltpu.async_copy(rows[b], acc_sh.at[dib[b]], ss[b], add=True)

    bl = (NFULL - 1) % 2
    pltpu.make_async_copy(rows[bl], acc_sh.at[dib[bl]], ss[bl]).wait()

    toff = NFULL * CH
    pltpu.async_copy(dst_hbm.at[pl.ds(base_e + toff, TAIL)], di_t, sem_t)
    pltpu.async_copy(attr_hbm.at[pl.ds(base_e + toff, TAIL)], av_t, sem_t)
    cpt = pltpu.async_copy(
        x_hbm.at[s_all.at[pl.ds(toff, TAIL)]], rows_t, sem_t)
    pltpu.make_async_copy(
        dst_hbm.at[pl.ds(base_e + toff, TAIL)], di_t, sem_t).wait()
    pltpu.make_async_copy(
        attr_hbm.at[pl.ds(base_e + toff, TAIL)], av_t, sem_t).wait()
    cpt.wait()
    scale_rows(rows_t, av_t, TAIL)
    pltpu.sync_copy(rows_t, acc_sh.at[di_t], add=True)

    plsc.subcore_barrier()
    pltpu.sync_copy(acc_sh.at[pl.ds(r0, ROWS_PER_TILE)],
                    part_hbm.at[cid, pl.ds(r0, ROWS_PER_TILE)])

  @functools.partial(
      pl.kernel,
      out_type=jax.ShapeDtypeStruct((N_PAD, D), jnp.float32),
      mesh=mesh,
      compiler_params=pltpu.CompilerParams(needs_layout_passes=False),
      scratch_types=[
          pltpu.VMEM((CR, D), jnp.float32),
          pltpu.VMEM((CR, D), jnp.float32),
      ],
  )
  def combine(part_hbm, out_hbm, a, b):
    base = _wid() * R_PER_W

    @pl.loop(0, R_PER_W // CR)
    def _chunks(c):
      o = base + c * CR
      pltpu.sync_copy(part_hbm.at[0, pl.ds(o, CR)], a)
      pltpu.sync_copy(part_hbm.at[1, pl.ds(o, CR)], b)

      @pl.loop(0, CR)
      def _relu(r):
        for dd in range(D // L):
          sl = pl.ds(dd * L, L)
          a[r, sl] = jnp.maximum(a[r, sl] + b[r, sl], 0.0)

      pltpu.sync_copy(a, out_hbm.at[pl.ds(o, CR)])

  @functools.partial(
      pl.kernel,
      out_type=jax.ShapeDtypeStruct((N_EDGES,), jnp.float32),
      mesh=mesh,
      compiler_params=pltpu.CompilerParams(needs_layout_passes=False),
      scratch_types=[
          pltpu.VMEM((E_PER_W,), jnp.int32),
          pltpu.VMEM((E_PER_W,), jnp.int32),
          pltpu.VMEM((E_PER_W,), jnp.float32),
          [pltpu.VMEM((CH, D), jnp.float32)] * NB,
          [pltpu.VMEM((CH, D), jnp.float32)] * NB,
          [pltpu.SemaphoreType.DMA] * NB,
          [pltpu.SemaphoreType.DMA] * NB,
      ],
  )
  def score(out_hbm, src_hbm, dst_hbm, score_hbm,
            s_all, d_all, sv, A, B, sa, sb):
    base_e = _wid() * E_PER_W
    lanes = lax.broadcasted_iota(jnp.int32, (L,), 0)

    pltpu.sync_copy(src_hbm.at[pl.ds(base_e, E_PER_W)], s_all)
    pltpu.sync_copy(dst_hbm.at[pl.ds(base_e, E_PER_W)], d_all)

    def off_of(k):
      return jnp.minimum(k * CH, E_PER_W - CH)

    def gathers(k, b):
      off = off_of(k)
      pltpu.async_copy(out_hbm.at[s_all.at[pl.ds(off, CH)]], A[b], sa[b])
      pltpu.async_copy(out_hbm.at[d_all.at[pl.ds(off, CH)]], B[b], sb[b])

    gathers(0, 0)
    gathers(1, 1)

    @pl.loop(0, NSC // NB)
    def _outer(i):
      for b in range(NB):
        k = i * NB + b
        off = off_of(k)

        @pl.when(k + 2 < NSC)
        def _refill():
          gathers(k + 2, (b + 2) % NB)

        pltpu.make_async_copy(
            out_hbm.at[s_all.at[pl.ds(off, CH)]], A[b], sa[b]).wait()
        pltpu.make_async_copy(
            out_hbm.at[d_all.at[pl.ds(off, CH)]], B[b], sb[b]).wait()

        @pl.loop(0, CH // L)
        def _group(g):
          rid = lanes + g * L
          zf = jnp.zeros((L,), jnp.float32)
          UN = 16

          @pl.loop(0, D // UN,
                   init_carry=(zf, zf, zf, zf, jnp.zeros((L,), jnp.int32)))
          def _dot(_, carry):
            a0, a1, a2, a3, colv = carry
            accs = [a0, a1, a2, a3]
            for u in range(UN):
              va = plsc.load_gather(A[b], [rid, colv])
              vb = plsc.load_gather(B[b], [rid, colv])
              accs[u % 4] = accs[u % 4] + va * vb
              colv = colv + 1
            return accs[0], accs[1], accs[2], accs[3], colv

          a0, a1, a2, a3, _ = _dot
          sv[pl.ds(off + g * L, L)] = (a0 + a1) + (a2 + a3)

    pltpu.sync_copy(sv, score_hbm.at[pl.ds(base_e, E_PER_W)])

  return accum, combine, score


def kernel(x, edge_index, edge_attr):
  accum, combine, score = _build()
  src = edge_index[0].astype(jnp.int32)
  dst = edge_index[1].astype(jnp.int32)
  attr = edge_attr.astype(jnp.float32)
  zeros = jnp.zeros((N_PAD, D), jnp.float32)
  part = accum(x, src, dst, attr, zeros)
  out = combine(part)
  return score(out, src, dst)


# speedup vs baseline: 4.2339x; 4.2339x over previous
import functools

import jax
import jax.numpy as jnp
from jax import lax
from jax.experimental import pallas as pl
from jax.experimental.pallas import tpu as pltpu
from jax.experimental.pallas import tpu_sc as plsc

N_NODES = 10000
N_PAD = 10240
N_EDGES = 320000
D = 128
L = 16
NC, NS = 2, 16
NW = NC * NS
E_PER_W = N_EDGES // NW
CH = 128
NB = 3
NFULL = E_PER_W // CH
TAIL = E_PER_W - NFULL * CH
ROWS_PER_TILE = N_PAD // NS
CR = 64
R_PER_W = N_PAD // NW
NSC = 81


def _wid():
  return lax.axis_index("s") * NC + lax.axis_index("c")


@functools.cache
def _build():
  mesh = plsc.VectorSubcoreMesh(
      core_axis_name="c", subcore_axis_name="s", num_cores=NC, num_subcores=NS)

  @functools.partial(
      pl.kernel,
      out_type=jax.ShapeDtypeStruct((NC, N_PAD, D), jnp.float32),
      mesh=mesh,
      compiler_params=pltpu.CompilerParams(needs_layout_passes=False),
      scratch_types=[
          pltpu.VMEM_SHARED((N_PAD, D), jnp.float32),
          pltpu.VMEM((E_PER_W,), jnp.int32),
          [pltpu.VMEM((CH, D), jnp.float32)] * 2,
          [pltpu.VMEM((CH,), jnp.int32)] * 2,
          [pltpu.VMEM((CH,), jnp.float32)] * 2,
          pltpu.VMEM((TAIL, D), jnp.float32),
          pltpu.VMEM((TAIL,), jnp.int32),
          pltpu.VMEM((TAIL,), jnp.float32),
          [pltpu.SemaphoreType.DMA] * 2,
          [pltpu.SemaphoreType.DMA] * 2,
          [pltpu.SemaphoreType.DMA] * 2,
          pltpu.SemaphoreType.DMA,
      ],
  )
  def accum(x_hbm, src_hbm, dst_hbm, attr_hbm, zero_hbm, part_hbm,
            acc_sh, s_all, rows, dib, av, rows_t, di_t, av_t,
            sg, sm, ss, sem_t):
    cid = lax.axis_index("c")
    sid = lax.axis_index("s")
    base_e = _wid() * E_PER_W

    r0 = sid * ROWS_PER_TILE
    pltpu.sync_copy(zero_hbm.at[pl.ds(r0, ROWS_PER_TILE)],
                    acc_sh.at[pl.ds(r0, ROWS_PER_TILE)])
    pltpu.sync_copy(src_hbm.at[pl.ds(base_e, E_PER_W)], s_all)
    plsc.subcore_barrier()

    def fetch(k, b):
      pltpu.async_copy(dst_hbm.at[pl.ds(base_e + k * CH, CH)], dib[b], sm[b])
      pltpu.async_copy(attr_hbm.at[pl.ds(base_e + k * CH, CH)], av[b], sm[b])
      pltpu.async_copy(x_hbm.at[s_all.at[pl.ds(k * CH, CH)]], rows[b], sg[b])

    def wait_fetch(k, b):
      pltpu.make_async_copy(
          dst_hbm.at[pl.ds(base_e + k * CH, CH)], dib[b], sm[b]).wait()
      pltpu.make_async_copy(
          attr_hbm.at[pl.ds(base_e + k * CH, CH)], av[b], sm[b]).wait()
      pltpu.make_async_copy(
          x_hbm.at[s_all.at[pl.ds(k * CH, CH)]], rows[b], sg[b]).wait()

    def scale_rows(buf, attr_buf, n):
      @pl.loop(0, n)
      def _scale(e):
        w = plsc.load_gather(attr_buf, [jnp.zeros((L,), jnp.int32) + e])
        for dd in range(D // L):
          sl = pl.ds(dd * L, L)
          buf[e, sl] = buf[e, sl] * w

    fetch(0, 0)

    @pl.loop(0, NFULL // 2)
    def _outer(i):
      for b in range(2):
        k = i * 2 + b
        bn = 1 - b

        @pl.when(k >= 1)
        def _retire():
          pltpu.make_async_copy(rows[bn], acc_sh.at[dib[bn]], ss[bn]).wait()

        @pl.when(k + 1 < NFULL)
        def _refill():
          fetch(k + 1, bn)

        wait_fetch(k, b)
        scale_rows(rows[b], av[b], CH)
        pltpu.async_copy(rows[b], acc_sh.at[dib[b]], ss[b], add=True)

    bl = (NFULL - 1) % 2
    pltpu.make_async_copy(rows[bl], acc_sh.at[dib[bl]], ss[bl]).wait()

    toff = NFULL * CH
    pltpu.async_copy(dst_hbm.at[pl.ds(base_e + toff, TAIL)], di_t, sem_t)
    pltpu.async_copy(attr_hbm.at[pl.ds(base_e + toff, TAIL)], av_t, sem_t)
    cpt = pltpu.async_copy(
        x_hbm.at[s_all.at[pl.ds(toff, TAIL)]], rows_t, sem_t)
    pltpu.make_async_copy(
        dst_hbm.at[pl.ds(base_e + toff, TAIL)], di_t, sem_t).wait()
    pltpu.make_async_copy(
        attr_hbm.at[pl.ds(base_e + toff, TAIL)], av_t, sem_t).wait()
    cpt.wait()
    scale_rows(rows_t, av_t, TAIL)
    pltpu.sync_copy(rows_t, acc_sh.at[di_t], add=True)

    plsc.subcore_barrier()
    pltpu.sync_copy(acc_sh.at[pl.ds(r0, ROWS_PER_TILE)],
                    part_hbm.at[cid, pl.ds(r0, ROWS_PER_TILE)])

  @functools.partial(
      pl.kernel,
      out_type=jax.ShapeDtypeStruct((N_PAD, D), jnp.float32),
      mesh=mesh,
      compiler_params=pltpu.CompilerParams(needs_layout_passes=False),
      scratch_types=[
          pltpu.VMEM((CR, D), jnp.float32),
          pltpu.VMEM((CR, D), jnp.float32),
      ],
  )
  def combine(part_hbm, out_hbm, a, b):
    base = _wid() * R_PER_W

    @pl.loop(0, R_PER_W // CR)
    def _chunks(c):
      o = base + c * CR
      pltpu.sync_copy(part_hbm.at[0, pl.ds(o, CR)], a)
      pltpu.sync_copy(part_hbm.at[1, pl.ds(o, CR)], b)

      @pl.loop(0, CR)
      def _relu(r):
        for dd in range(D // L):
          sl = pl.ds(dd * L, L)
          a[r, sl] = jnp.maximum(a[r, sl] + b[r, sl], 0.0)

      pltpu.sync_copy(a, out_hbm.at[pl.ds(o, CR)])

  @functools.partial(
      pl.kernel,
      out_type=jax.ShapeDtypeStruct((N_EDGES,), jnp.float32),
      mesh=mesh,
      compiler_params=pltpu.CompilerParams(needs_layout_passes=False),
      scratch_types=[
          pltpu.VMEM((E_PER_W,), jnp.int32),
          pltpu.VMEM((E_PER_W,), jnp.int32),
          pltpu.VMEM((E_PER_W,), jnp.float32),
          [pltpu.VMEM((CH, D), jnp.float32)] * NB,
          [pltpu.VMEM((CH, D), jnp.float32)] * NB,
          [pltpu.SemaphoreType.DMA] * NB,
          [pltpu.SemaphoreType.DMA] * NB,
      ],
  )
  def score(out_hbm, src_hbm, dst_hbm, score_hbm,
            s_all, d_all, sv, A, B, sa, sb):
    base_e = _wid() * E_PER_W
    lanes = lax.broadcasted_iota(jnp.int32, (L,), 0)

    pltpu.sync_copy(src_hbm.at[pl.ds(base_e, E_PER_W)], s_all)
    pltpu.sync_copy(dst_hbm.at[pl.ds(base_e, E_PER_W)], d_all)

    def off_of(k):
      return jnp.minimum(k * CH, E_PER_W - CH)

    def gathers(k, b):
      off = off_of(k)
      pltpu.async_copy(out_hbm.at[s_all.at[pl.ds(off, CH)]], A[b], sa[b])
      pltpu.async_copy(out_hbm.at[d_all.at[pl.ds(off, CH)]], B[b], sb[b])

    gathers(0, 0)
    gathers(1, 1)

    @pl.loop(0, NSC // NB)
    def _outer(i):
      for b in range(NB):
        k = i * NB + b
        off = off_of(k)

        @pl.when(k + 2 < NSC)
        def _refill():
          gathers(k + 2, (b + 2) % NB)

        pltpu.make_async_copy(
            out_hbm.at[s_all.at[pl.ds(off, CH)]], A[b], sa[b]).wait()
        pltpu.make_async_copy(
            out_hbm.at[d_all.at[pl.ds(off, CH)]], B[b], sb[b]).wait()

        @pl.loop(0, CH // L)
        def _group(g):
          rid = lanes + g * L
          zf = jnp.zeros((L,), jnp.float32)
          UN = 16

          @pl.loop(0, D // UN,
                   init_carry=(zf, zf, zf, zf, jnp.zeros((L,), jnp.int32)))
          def _dot(_, carry):
            a0, a1, a2, a3, dv = carry
            accs = [a0, a1, a2, a3]
            for u in range(UN):
              col = (dv + u) ^ lanes
              va = plsc.load_gather(A[b], [rid, col])
              vb = plsc.load_gather(B[b], [rid, col])
              accs[u % 4] = accs[u % 4] + va * vb
            return accs[0], accs[1], accs[2], accs[3], dv + UN

          a0, a1, a2, a3, _ = _dot
          sv[pl.ds(off + g * L, L)] = (a0 + a1) + (a2 + a3)

    pltpu.sync_copy(sv, score_hbm.at[pl.ds(base_e, E_PER_W)])

  return accum, combine, score


def kernel(x, edge_index, edge_attr):
  accum, combine, score = _build()
  src = edge_index[0].astype(jnp.int32)
  dst = edge_index[1].astype(jnp.int32)
  attr = edge_attr.astype(jnp.float32)
  zeros = jnp.zeros((N_PAD, D), jnp.float32)
  part = accum(x, src, dst, attr, zeros)
  out = combine(part)
  return score(out, src, dst)


# speedup vs baseline: 4.7678x; 1.1261x over previous
import functools

import jax
import jax.numpy as jnp
from jax import lax
from jax.experimental import pallas as pl
from jax.experimental.pallas import tpu as pltpu
from jax.experimental.pallas import tpu_sc as plsc

N_NODES = 10000
N_PAD = 10240
N_EDGES = 320000
D = 128
L = 16
NC, NS = 2, 16
NW = NC * NS
E_PER_W = N_EDGES // NW
CH = 128
NB = 3
NFULL = E_PER_W // CH
TAIL = E_PER_W - NFULL * CH
ROWS_PER_TILE = N_PAD // NS
CR = 64
R_PER_W = N_PAD // NW
NSC = 81


def _wid():
  return lax.axis_index("s") * NC + lax.axis_index("c")


@functools.cache
def _build():
  mesh = plsc.VectorSubcoreMesh(
      core_axis_name="c", subcore_axis_name="s", num_cores=NC, num_subcores=NS)

  @functools.partial(
      pl.kernel,
      out_type=jax.ShapeDtypeStruct((NC, N_PAD, D), jnp.float32),
      mesh=mesh,
      compiler_params=pltpu.CompilerParams(needs_layout_passes=False),
      scratch_types=[
          pltpu.VMEM_SHARED((N_PAD, D), jnp.float32),
          pltpu.VMEM((E_PER_W,), jnp.int32),
          [pltpu.VMEM((CH, D), jnp.float32)] * 2,
          [pltpu.VMEM((CH,), jnp.int32)] * 2,
          [pltpu.VMEM((CH,), jnp.float32)] * 2,
          pltpu.VMEM((TAIL, D), jnp.float32),
          pltpu.VMEM((TAIL,), jnp.int32),
          pltpu.VMEM((TAIL,), jnp.float32),
          [pltpu.SemaphoreType.DMA] * 2,
          [pltpu.SemaphoreType.DMA] * 2,
          [pltpu.SemaphoreType.DMA] * 2,
          pltpu.SemaphoreType.DMA,
      ],
  )
  def accum(x_hbm, src_hbm, dst_hbm, attr_hbm, zero_hbm, part_hbm,
            acc_sh, s_all, rows, dib, av, rows_t, di_t, av_t,
            sg, sm, ss, sem_t):
    cid = lax.axis_index("c")
    sid = lax.axis_index("s")
    base_e = _wid() * E_PER_W

    r0 = sid * ROWS_PER_TILE
    pltpu.sync_copy(zero_hbm.at[pl.ds(r0, ROWS_PER_TILE)],
                    acc_sh.at[pl.ds(r0, ROWS_PER_TILE)])
    pltpu.sync_copy(src_hbm.at[pl.ds(base_e, E_PER_W)], s_all)
    plsc.subcore_barrier()

    def fetch(k, b):
      pltpu.async_copy(dst_hbm.at[pl.ds(base_e + k * CH, CH)], dib[b], sm[b])
      pltpu.async_copy(attr_hbm.at[pl.ds(base_e + k * CH, CH)], av[b], sm[b])
      pltpu.async_copy(x_hbm.at[s_all.at[pl.ds(k * CH, CH)]], rows[b], sg[b])

    def wait_fetch(k, b):
      pltpu.make_async_copy(
          dst_hbm.at[pl.ds(base_e + k * CH, CH)], dib[b], sm[b]).wait()
      pltpu.make_async_copy(
          attr_hbm.at[pl.ds(base_e + k * CH, CH)], av[b], sm[b]).wait()
      pltpu.make_async_copy(
          x_hbm.at[s_all.at[pl.ds(k * CH, CH)]], rows[b], sg[b]).wait()

    def scale_rows(buf, attr_buf, n):
      @pl.loop(0, n // L)
      def _scale(g):
        w16 = attr_buf[pl.ds(g * L, L)]
        for r in range(L):
          e = g * L + r
          w = jnp.broadcast_to(w16[r], (L,))
          for dd in range(D // L):
            sl = pl.ds(dd * L, L)
            buf[e, sl] = buf[e, sl] * w

    fetch(0, 0)

    @pl.loop(0, NFULL // 2)
    def _outer(i):
      for b in range(2):
        k = i * 2 + b
        bn = 1 - b

        @pl.when(k >= 1)
        def _retire():
          pltpu.make_async_copy(rows[bn], acc_sh.at[dib[bn]], ss[bn]).wait()

        @pl.when(k + 1 < NFULL)
        def _refill():
          fetch(k + 1, bn)

        wait_fetch(k, b)
        scale_rows(rows[b], av[b], CH)
        pltpu.async_copy(rows[b], acc_sh.at[dib[b]], ss[b], add=True)

    bl = (NFULL - 1) % 2
    pltpu.make_async_copy(rows[bl], acc_sh.at[dib[bl]], ss[bl]).wait()

    toff = NFULL * CH
    pltpu.async_copy(dst_hbm.at[pl.ds(base_e + toff, TAIL)], di_t, sem_t)
    pltpu.async_copy(attr_hbm.at[pl.ds(base_e + toff, TAIL)], av_t, sem_t)
    cpt = pltpu.async_copy(
        x_hbm.at[s_all.at[pl.ds(toff, TAIL)]], rows_t, sem_t)
    pltpu.make_async_copy(
        dst_hbm.at[pl.ds(base_e + toff, TAIL)], di_t, sem_t).wait()
    pltpu.make_async_copy(
        attr_hbm.at[pl.ds(base_e + toff, TAIL)], av_t, sem_t).wait()
    cpt.wait()
    scale_rows(rows_t, av_t, TAIL)
    pltpu.sync_copy(rows_t, acc_sh.at[di_t], add=True)

    plsc.subcore_barrier()
    pltpu.sync_copy(acc_sh.at[pl.ds(r0, ROWS_PER_TILE)],
                    part_hbm.at[cid, pl.ds(r0, ROWS_PER_TILE)])

  @functools.partial(
      pl.kernel,
      out_type=jax.ShapeDtypeStruct((N_PAD, D), jnp.float32),
      mesh=mesh,
      compiler_params=pltpu.CompilerParams(needs_layout_passes=False),
      scratch_types=[
          pltpu.VMEM((CR, D), jnp.float32),
          pltpu.VMEM((CR, D), jnp.float32),
      ],
  )
  def combine(part_hbm, out_hbm, a, b):
    base = _wid() * R_PER_W

    @pl.loop(0, R_PER_W // CR)
    def _chunks(c):
      o = base + c * CR
      pltpu.sync_copy(part_hbm.at[0, pl.ds(o, CR)], a)
      pltpu.sync_copy(part_hbm.at[1, pl.ds(o, CR)], b)

      @pl.loop(0, CR)
      def _relu(r):
        for dd in range(D // L):
          sl = pl.ds(dd * L, L)
          a[r, sl] = jnp.maximum(a[r, sl] + b[r, sl], 0.0)

      pltpu.sync_copy(a, out_hbm.at[pl.ds(o, CR)])

  @functools.partial(
      pl.kernel,
      out_type=jax.ShapeDtypeStruct((N_EDGES,), jnp.float32),
      mesh=mesh,
      compiler_params=pltpu.CompilerParams(needs_layout_passes=False),
      scratch_types=[
          pltpu.VMEM((E_PER_W,), jnp.int32),
          pltpu.VMEM((E_PER_W,), jnp.int32),
          pltpu.VMEM((E_PER_W,), jnp.float32),
          [pltpu.VMEM((CH, D), jnp.float32)] * NB,
          [pltpu.VMEM((CH, D), jnp.float32)] * NB,
          [pltpu.SemaphoreType.DMA] * NB,
          [pltpu.SemaphoreType.DMA] * NB,
      ],
  )
  def score(out_hbm, src_hbm, dst_hbm, score_hbm,
            s_all, d_all, sv, A, B, sa, sb):
    base_e = _wid() * E_PER_W
    lanes = lax.broadcasted_iota(jnp.int32, (L,), 0)

    pltpu.sync_copy(src_hbm.at[pl.ds(base_e, E_PER_W)], s_all)
    pltpu.sync_copy(dst_hbm.at[pl.ds(base_e, E_PER_W)], d_all)

    def off_of(k):
      return jnp.minimum(k * CH, E_PER_W - CH)

    def gathers(k, b):
      off = off_of(k)
      pltpu.async_copy(out_hbm.at[s_all.at[pl.ds(off, CH)]], A[b], sa[b])
      pltpu.async_copy(out_hbm.at[d_all.at[pl.ds(off, CH)]], B[b], sb[b])

    gathers(0, 0)
    gathers(1, 1)

    @pl.loop(0, NSC // NB)
    def _outer(i):
      for b in range(NB):
        k = i * NB + b
        off = off_of(k)

        @pl.when(k + 2 < NSC)
        def _refill():
          gathers(k + 2, (b + 2) % NB)

        pltpu.make_async_copy(
            out_hbm.at[s_all.at[pl.ds(off, CH)]], A[b], sa[b]).wait()
        pltpu.make_async_copy(
            out_hbm.at[d_all.at[pl.ds(off, CH)]], B[b], sb[b]).wait()

        @pl.loop(0, CH // L)
        def _group(g):
          rid = lanes + g * L
          zf = jnp.zeros((L,), jnp.float32)
          UN = 16

          @pl.loop(0, D // UN,
                   init_carry=(zf, zf, zf, zf, jnp.zeros((L,), jnp.int32)))
          def _dot(_, carry):
            a0, a1, a2, a3, dv = carry
            accs = [a0, a1, a2, a3]
            for u in range(UN):
              col = (dv + u) ^ lanes
              va = plsc.load_gather(A[b], [rid, col])
              vb = plsc.load_gather(B[b], [rid, col])
              accs[u % 4] = accs[u % 4] + va * vb
            return accs[0], accs[1], accs[2], accs[3], dv + UN

          a0, a1, a2, a3, _ = _dot
          sv[pl.ds(off + g * L, L)] = (a0 + a1) + (a2 + a3)

    pltpu.sync_copy(sv, score_hbm.at[pl.ds(base_e, E_PER_W)])

  return accum, combine, score


def kernel(x, edge_index, edge_attr):
  accum, combine, score = _build()
  src = edge_index[0].astype(jnp.int32)
  dst = edge_index[1].astype(jnp.int32)
  attr = edge_attr.astype(jnp.float32)
  zeros = jnp.zeros((N_PAD, D), jnp.float32)
  part = accum(x, src, dst, attr, zeros)
  out = combine(part)
  return score(out, src, dst)


# speedup vs baseline: 5.2296x; 1.0969x over previous
import functools

import jax
import jax.numpy as jnp
from jax import lax
from jax.experimental import pallas as pl
from jax.experimental.pallas import tpu as pltpu
from jax.experimental.pallas import tpu_sc as plsc

N_NODES = 10000
N_PAD = 10240
N_EDGES = 320000
D = 128
L = 16
NC, NS = 2, 16
NW = NC * NS
E_PER_W = N_EDGES // NW
CH = 128
NB = 3
NFULL = E_PER_W // CH
TAIL = E_PER_W - NFULL * CH
ROWS_PER_TILE = N_PAD // NS
CR = 64
R_PER_W = N_PAD // NW
NSC = 81


def _wid():
  return lax.axis_index("s") * NC + lax.axis_index("c")


@functools.cache
def _build():
  mesh = plsc.VectorSubcoreMesh(
      core_axis_name="c", subcore_axis_name="s", num_cores=NC, num_subcores=NS)

  @functools.partial(
      pl.kernel,
      out_type=jax.ShapeDtypeStruct((NC, N_PAD, D), jnp.float32),
      mesh=mesh,
      compiler_params=pltpu.CompilerParams(needs_layout_passes=False, use_tc_tiling_on_sc=False),
      scratch_types=[
          pltpu.VMEM_SHARED((N_PAD, D), jnp.float32),
          pltpu.VMEM((E_PER_W,), jnp.int32),
          [pltpu.VMEM((CH, D), jnp.float32)] * 2,
          [pltpu.VMEM((CH,), jnp.int32)] * 2,
          [pltpu.VMEM((CH,), jnp.float32)] * 2,
          pltpu.VMEM((TAIL, D), jnp.float32),
          pltpu.VMEM((TAIL,), jnp.int32),
          pltpu.VMEM((TAIL,), jnp.float32),
          [pltpu.SemaphoreType.DMA] * 2,
          [pltpu.SemaphoreType.DMA] * 2,
          [pltpu.SemaphoreType.DMA] * 2,
          pltpu.SemaphoreType.DMA,
      ],
  )
  def accum(x_hbm, src_hbm, dst_hbm, attr_hbm, zero_hbm, part_hbm,
            acc_sh, s_all, rows, dib, av, rows_t, di_t, av_t,
            sg, sm, ss, sem_t):
    cid = lax.axis_index("c")
    sid = lax.axis_index("s")
    base_e = _wid() * E_PER_W

    r0 = sid * ROWS_PER_TILE
    pltpu.sync_copy(zero_hbm.at[pl.ds(r0, ROWS_PER_TILE)],
                    acc_sh.at[pl.ds(r0, ROWS_PER_TILE)])
    pltpu.sync_copy(src_hbm.at[pl.ds(base_e, E_PER_W)], s_all)
    plsc.subcore_barrier()

    def fetch(k, b):
      pltpu.async_copy(dst_hbm.at[pl.ds(base_e + k * CH, CH)], dib[b], sm[b])
      pltpu.async_copy(attr_hbm.at[pl.ds(base_e + k * CH, CH)], av[b], sm[b])
      pltpu.async_copy(x_hbm.at[s_all.at[pl.ds(k * CH, CH)]], rows[b], sg[b])

    def wait_fetch(k, b):
      pltpu.make_async_copy(
          dst_hbm.at[pl.ds(base_e + k * CH, CH)], dib[b], sm[b]).wait()
      pltpu.make_async_copy(
          attr_hbm.at[pl.ds(base_e + k * CH, CH)], av[b], sm[b]).wait()
      pltpu.make_async_copy(
          x_hbm.at[s_all.at[pl.ds(k * CH, CH)]], rows[b], sg[b]).wait()

    def scale_rows(buf, attr_buf, n):
      @pl.loop(0, n // L)
      def _scale(g):
        w16 = attr_buf[pl.ds(g * L, L)]
        for r in range(L):
          e = g * L + r
          w = jnp.broadcast_to(w16[r], (L,))
          for dd in range(D // L):
            sl = pl.ds(dd * L, L)
            buf[e, sl] = buf[e, sl] * w

    fetch(0, 0)

    @pl.loop(0, NFULL // 2)
    def _outer(i):
      for b in range(2):
        k = i * 2 + b
        bn = 1 - b

        @pl.when(k >= 1)
        def _retire():
          pltpu.make_async_copy(rows[bn], acc_sh.at[dib[bn]], ss[bn]).wait()

        @pl.when(k + 1 < NFULL)
        def _refill():
          fetch(k + 1, bn)

        wait_fetch(k, b)
        scale_rows(rows[b], av[b], CH)
        pltpu.async_copy(rows[b], acc_sh.at[dib[b]], ss[b], add=True)

    bl = (NFULL - 1) % 2
    pltpu.make_async_copy(rows[bl], acc_sh.at[dib[bl]], ss[bl]).wait()

    toff = NFULL * CH
    pltpu.async_copy(dst_hbm.at[pl.ds(base_e + toff, TAIL)], di_t, sem_t)
    pltpu.async_copy(attr_hbm.at[pl.ds(base_e + toff, TAIL)], av_t, sem_t)
    cpt = pltpu.async_copy(
        x_hbm.at[s_all.at[pl.ds(toff, TAIL)]], rows_t, sem_t)
    pltpu.make_async_copy(
        dst_hbm.at[pl.ds(base_e + toff, TAIL)], di_t, sem_t).wait()
    pltpu.make_async_copy(
        attr_hbm.at[pl.ds(base_e + toff, TAIL)], av_t, sem_t).wait()
    cpt.wait()
    scale_rows(rows_t, av_t, TAIL)
    pltpu.sync_copy(rows_t, acc_sh.at[di_t], add=True)

    plsc.subcore_barrier()
    pltpu.sync_copy(acc_sh.at[pl.ds(r0, ROWS_PER_TILE)],
                    part_hbm.at[cid, pl.ds(r0, ROWS_PER_TILE)])

  @functools.partial(
      pl.kernel,
      out_type=jax.ShapeDtypeStruct((N_PAD, D // 2), jnp.int32),
      mesh=mesh,
      compiler_params=pltpu.CompilerParams(needs_layout_passes=False, use_tc_tiling_on_sc=False),
      scratch_types=[
          pltpu.VMEM((CR, D), jnp.float32),
          pltpu.VMEM((CR, D), jnp.float32),
          pltpu.VMEM((CR, D // 2), jnp.int32),
      ],
  )
  def combine(part_hbm, out_hbm, a, b, o):
    base = _wid() * R_PER_W

    @pl.loop(0, R_PER_W // CR)
    def _chunks(c):
      ro = base + c * CR
      pltpu.sync_copy(part_hbm.at[0, pl.ds(ro, CR)], a)
      pltpu.sync_copy(part_hbm.at[1, pl.ds(ro, CR)], b)

      @pl.loop(0, CR)
      def _relu(r):
        for h in range(D // (2 * L)):
          c0 = pl.ds(2 * h * L, L)
          c1 = pl.ds((2 * h + 1) * L, L)
          v0 = jnp.maximum(a[r, c0] + b[r, c0], 0.0)
          v1 = jnp.maximum(a[r, c1] + b[r, c1], 0.0)
          pk = plsc.pack(v0, v1, format=plsc.PackFormat.INTERLEAVED)
          o[r, pl.ds(h * L, L)] = plsc.bitcast(pk, jnp.int32)

      pltpu.sync_copy(o, out_hbm.at[pl.ds(ro, CR)])

  @functools.partial(
      pl.kernel,
      out_type=jax.ShapeDtypeStruct((N_EDGES,), jnp.float32),
      mesh=mesh,
      compiler_params=pltpu.CompilerParams(needs_layout_passes=False, use_tc_tiling_on_sc=False),
      scratch_types=[
          pltpu.VMEM((E_PER_W,), jnp.int32),
          pltpu.VMEM((E_PER_W,), jnp.int32),
          pltpu.VMEM((E_PER_W,), jnp.float32),
          [pltpu.VMEM((CH, D // 2), jnp.int32)] * NB,
          [pltpu.VMEM((CH, D // 2), jnp.int32)] * NB,
          [pltpu.SemaphoreType.DMA] * NB,
          [pltpu.SemaphoreType.DMA] * NB,
      ],
  )
  def score(out_hbm, src_hbm, dst_hbm, score_hbm,
            s_all, d_all, sv, A, B, sa, sb):
    base_e = _wid() * E_PER_W
    lanes = lax.broadcasted_iota(jnp.int32, (L,), 0)

    pltpu.sync_copy(src_hbm.at[pl.ds(base_e, E_PER_W)], s_all)
    pltpu.sync_copy(dst_hbm.at[pl.ds(base_e, E_PER_W)], d_all)

    def off_of(k):
      return jnp.minimum(k * CH, E_PER_W - CH)

    def gathers(k, b):
      off = off_of(k)
      pltpu.async_copy(out_hbm.at[s_all.at[pl.ds(off, CH)]], A[b], sa[b])
      pltpu.async_copy(out_hbm.at[d_all.at[pl.ds(off, CH)]], B[b], sb[b])

    gathers(0, 0)
    gathers(1, 1)

    @pl.loop(0, NSC // NB)
    def _outer(i):
      for b in range(NB):
        k = i * NB + b
        off = off_of(k)

        @pl.when(k + 2 < NSC)
        def _refill():
          gathers(k + 2, (b + 2) % NB)

        pltpu.make_async_copy(
            out_hbm.at[s_all.at[pl.ds(off, CH)]], A[b], sa[b]).wait()
        pltpu.make_async_copy(
            out_hbm.at[d_all.at[pl.ds(off, CH)]], B[b], sb[b]).wait()

        @pl.loop(0, CH // L)
        def _group(g):
          rid = lanes + g * L
          zf = jnp.zeros((L,), jnp.float32)
          UN = 16

          @pl.loop(0, (D // 2) // UN,
                   init_carry=(zf, zf, zf, zf, jnp.zeros((L,), jnp.int32)))
          def _dot(_, carry):
            a0, a1, a2, a3, dv = carry
            accs = [a0, a1, a2, a3]
            for u in range(UN):
              col = (dv + u) ^ lanes
              va = plsc.load_gather(A[b], [rid, col])
              vb = plsc.load_gather(B[b], [rid, col])
              wa0, wa1 = plsc.unpack(
                  plsc.bitcast(va, jnp.bfloat16),
                  format=plsc.PackFormat.INTERLEAVED)
              wb0, wb1 = plsc.unpack(
                  plsc.bitcast(vb, jnp.bfloat16),
                  format=plsc.PackFormat.INTERLEAVED)
              accs[u % 4] = accs[u % 4] + (wa0 * wb0 + wa1 * wb1)
            return accs[0], accs[1], accs[2], accs[3], dv + UN

          a0, a1, a2, a3, _ = _dot
          sv[pl.ds(off + g * L, L)] = (a0 + a1) + (a2 + a3)

    pltpu.sync_copy(sv, score_hbm.at[pl.ds(base_e, E_PER_W)])

  return accum, combine, score


def kernel(x, edge_index, edge_attr):
  accum, combine, score = _build()
  src = edge_index[0].astype(jnp.int32)
  dst = edge_index[1].astype(jnp.int32)
  attr = edge_attr.astype(jnp.float32)
  zeros = jnp.zeros((N_PAD, D), jnp.float32)
  part = accum(x, src, dst, attr, zeros)
  out = combine(part)
  return score(out, src, dst)
